# baseline (speedup 1.0000x reference)
import functools

import jax
import jax.numpy as jnp
from jax import lax
from jax.experimental import pallas as pl
from jax.experimental.pallas import tpu as pltpu
from jax.experimental.pallas import tpu_sc as plsc

_N_ROWS = 4096 * 200
_D = 128
_NC, _NS = 2, 16
_NW = _NC * _NS
_ROWS_PER_W = _N_ROWS // _NW
_CHUNK = 128
_NIT = _ROWS_PER_W // _CHUNK
_NB = 4
_K = 128
_NSL = _ROWS_PER_W // 16
_GRP = _K // 16
_TROWS = 3 * _K
_STG = _TROWS // _NS


_mesh = plsc.VectorSubcoreMesh(core_axis_name="c", subcore_axis_name="s")


@functools.partial(
    pl.kernel,
    mesh=_mesh,
    out_type=jax.ShapeDtypeStruct((_N_ROWS, _D), jnp.float32),
    scratch_types=[
        pltpu.VMEM((_ROWS_PER_W,), jnp.int32),
        pltpu.VMEM((_NB, _CHUNK, _D), jnp.float32),
        pltpu.VMEM_SHARED((_TROWS, _D), jnp.float32),
        pltpu.SemaphoreType.DMA,
        pltpu.SemaphoreType.DMA,
        pltpu.SemaphoreType.DMA,
        pltpu.SemaphoreType.DMA,
        pltpu.SemaphoreType.DMA,
        pltpu.SemaphoreType.DMA,
        pltpu.SemaphoreType.DMA,
        pltpu.SemaphoreType.DMA,
    ],
)
def _gather_rows(idx_hbm, table_hbm, out_hbm, idx_v, rows_v, table_sh, *sems):
    gsem = sems[:_NB]
    ssem = sems[_NB:]
    sid = lax.axis_index("s")
    wid = sid * _NC + lax.axis_index("c")
    base = wid * _ROWS_PER_W
    pltpu.sync_copy(
        table_hbm.at[pl.ds(sid * _STG, _STG)],
        table_sh.at[pl.ds(sid * _STG, _STG)],
    )
    pltpu.sync_copy(idx_hbm.at[pl.ds(base, _ROWS_PER_W)], idx_v)

    plsc.subcore_barrier()

    lane_off = 3 * lax.iota(jnp.int32, 16)

    def fire_gather(it, b, src=None):
        rep = lane_off + jnp.full(
            (16,), 48 * lax.rem(it + wid, _GRP), dtype=jnp.int32
        )
        for u in range(_CHUNK // 16):
            sl = pl.ds(it * _CHUNK + u * 16, 16)
            idx_v[sl] = idx_v[sl] + rep
        pltpu.async_copy(
            (table_sh if src is None else src).at[
                idx_v.at[pl.ds(it * _CHUNK, _CHUNK)]
            ],
            rows_v.at[b],
            gsem[b],
        )

    def wait_gather(b):
        pltpu.make_async_copy(
            out_hbm.at[pl.ds(0, _CHUNK)], rows_v.at[b], gsem[b]
        ).wait()

    def fire_store(it, b):
        pltpu.async_copy(
            rows_v.at[b], out_hbm.at[pl.ds(base + it * _CHUNK, _CHUNK)], ssem[b]
        )

    def wait_store(b):
        pltpu.make_async_copy(
            rows_v.at[b], out_hbm.at[pl.ds(0, _CHUNK)], ssem[b]
        ).wait()

    srcs = (table_sh, table_hbm)

    fire_gather(0, 0, srcs[0])
    for it in range(_NB):
        b, b1 = it % _NB, (it + 1) % _NB
        if it + 1 - _NB >= 0:
            wait_store(b1)
        fire_gather(it + 1, b1, srcs[(it + 1) % 2])
        wait_gather(b)
        fire_store(it, b)

    def quad(q, _):
        for u in range(_NB):
            it = q * _NB + u
            b, b1 = u, (u + 1) % _NB
            wait_store(b1)
            fire_gather(it + 1, b1, srcs[(u + 1) % 2])
            wait_gather(b)
            fire_store(it, b)
        return ()

    lax.fori_loop(1, _NIT // _NB - 1, quad, ())

    for u in range(_NB):
        it = _NIT - _NB + u
        b, b1 = u, (u + 1) % _NB
        if it < _NIT - 1:
            wait_store(b1)
            fire_gather(it + 1, b1, srcs[(it + 1) % 2])
        wait_gather(b)
        fire_store(it, b)
    for b in range(_NB):
        wait_store(b)


def kernel(x, table):
    idx = x.reshape(_N_ROWS).astype(jnp.int32)
    rep_table = jnp.tile(table, (_K, 1))
    out = _gather_rows(idx, rep_table)
    return out.reshape(x.shape[0], x.shape[1], _D)


# speedup vs baseline: 2.8973x; 2.8973x over previous
import functools

import jax
import jax.numpy as jnp
from jax import lax
from jax.experimental import pallas as pl
from jax.experimental.pallas import tpu as pltpu
from jax.experimental.pallas import tpu_sc as plsc

_N_ROWS = 4096 * 200
_D = 128
_NC, _NS = 2, 16
_NW = _NC * _NS
_ROWS_PER_W = _N_ROWS // _NW
_CHUNK = 128
_NIT = _ROWS_PER_W // _CHUNK
_NB = 4
_K = 128
_NSL = _ROWS_PER_W // 16
_GRP = _K // 16
_TROWS = 3 * _K
_STG = _TROWS // _NS


_mesh = plsc.VectorSubcoreMesh(core_axis_name="c", subcore_axis_name="s")


@functools.partial(
    pl.kernel,
    mesh=_mesh,
    out_type=jax.ShapeDtypeStruct((_N_ROWS, _D), jnp.float32),
    scratch_types=[
        pltpu.VMEM((_ROWS_PER_W,), jnp.int32),
        pltpu.VMEM((_NB, _CHUNK, _D), jnp.float32),
        pltpu.VMEM_SHARED((_TROWS, _D), jnp.float32),
        pltpu.SemaphoreType.DMA,
        pltpu.SemaphoreType.DMA,
        pltpu.SemaphoreType.DMA,
        pltpu.SemaphoreType.DMA,
        pltpu.SemaphoreType.DMA,
        pltpu.SemaphoreType.DMA,
        pltpu.SemaphoreType.DMA,
        pltpu.SemaphoreType.DMA,
    ],
)
def _gather_rows(idx_hbm, table_hbm, out_hbm, idx_v, rows_v, table_sh, *sems):
    gsem = sems[:_NB]
    ssem = sems[_NB:]
    sid = lax.axis_index("s")
    wid = sid * _NC + lax.axis_index("c")
    base = wid * _ROWS_PER_W
    pltpu.sync_copy(
        table_hbm.at[pl.ds(sid * _STG, _STG)],
        table_sh.at[pl.ds(sid * _STG, _STG)],
    )
    pltpu.sync_copy(idx_hbm.at[pl.ds(base, _ROWS_PER_W)], idx_v)

    plsc.subcore_barrier()

    lane_off = 3 * lax.iota(jnp.int32, 16)

    def fire_gather(it, b):
        for u in range(_CHUNK // 16):
            sl = pl.ds(it * _CHUNK + u * 16, 16)
            idx_v[sl] = idx_v[sl] + lane_off
        pltpu.async_copy(
            table_sh.at[idx_v.at[pl.ds(it * _CHUNK, _CHUNK)]],
            rows_v.at[b],
            gsem[b],
        )

    def wait_gather(b):
        pltpu.make_async_copy(
            out_hbm.at[pl.ds(0, _CHUNK)], rows_v.at[b], gsem[b]
        ).wait()

    def fire_store(it, b):
        pltpu.async_copy(
            rows_v.at[b], out_hbm.at[pl.ds(base + it * _CHUNK, _CHUNK)], ssem[b]
        )

    def wait_store(b):
        pltpu.make_async_copy(
            rows_v.at[b], out_hbm.at[pl.ds(0, _CHUNK)], ssem[b]
        ).wait()

    fire_gather(0, 0)
    for it in range(_NB):
        b, b1 = it % _NB, (it + 1) % _NB
        if it + 1 - _NB >= 0:
            wait_store(b1)
        fire_gather(it + 1, b1)
        wait_gather(b)
        fire_store(it, b)

    def quad(q, _):
        for u in range(_NB):
            it = q * _NB + u
            b, b1 = u, (u + 1) % _NB
            wait_store(b1)
            fire_gather(it + 1, b1)
            wait_gather(b)
            fire_store(it, b)
        return ()

    lax.fori_loop(1, _NIT // _NB - 1, quad, ())

    for u in range(_NB):
        it = _NIT - _NB + u
        b, b1 = u, (u + 1) % _NB
        if it < _NIT - 1:
            wait_store(b1)
            fire_gather(it + 1, b1)
        wait_gather(b)
        fire_store(it, b)
    for b in range(_NB):
        wait_store(b)


def kernel(x, table):
    idx = x.reshape(_N_ROWS).astype(jnp.int32)
    rep_table = jnp.tile(table, (_K, 1))
    out = _gather_rows(idx, rep_table)
    return out.reshape(x.shape[0], x.shape[1], _D)
